# baseline (speedup 1.0000x reference)
import functools

import jax
import jax.numpy as jnp
from jax import lax
from jax.experimental import pallas as pl
from jax.experimental.pallas import tpu as pltpu
from jax.experimental.pallas import tpu_sc as plsc

N_USERS = 25000
N_ITEMS = 25000
D = 64
H = 32
NNZ = 800000
B = 1024
P = 25600
NP = 2 * P
N_SUB = 16
CH = 128
CPS = 20
NSG = 20
CPT = CPS * NSG
NNZP = N_SUB * CPT * CH
SLAB = NP // N_SUB
NRING = 5

_mesh = plsc.VectorSubcoreMesh(core_axis_name="c", subcore_axis_name="s")


@functools.partial(
    pl.kernel,
    out_type=(jax.ShapeDtypeStruct((NP, H), jnp.float32),
              jax.ShapeDtypeStruct((NP, H), jnp.float32)),
    mesh=_mesh,
    scratch_types=[
        pltpu.VMEM((CPS, 3, CH), jnp.int32),
        pltpu.VMEM((CH, H), jnp.float32),
        pltpu.VMEM((CH, H), jnp.float32),
        pltpu.VMEM((CH, H), jnp.float32),
        pltpu.VMEM((CH, H), jnp.float32),
        pltpu.VMEM((CH, H), jnp.float32),
        pltpu.VMEM_SHARED((NP, H), jnp.float32),
        pltpu.SemaphoreType.DMA((NRING,)),
        pltpu.SemaphoreType.DMA((NRING,)),
    ],
    compiler_params=pltpu.CompilerParams(needs_layout_passes=False,
                                         use_tc_tiling_on_sc=False),
)
def _spmm(ego_l, ego_r, packed, zpage, nb_l, nb_r,
          pk_v, rb0, rb1, rb2, rb3, rb4, acc, gsem, ssem):
    c = lax.axis_index("c")
    s = lax.axis_index("s")
    rbufs = (rb0, rb1, rb2, rb3, rb4)

    def gather_start(j, k):
        idx = pk_v.at[j, 0]

        @pl.when(c == 0)
        def _():
            pltpu.async_copy(ego_l.at[idx], rbufs[k], gsem.at[k])

        @pl.when(c == 1)
        def _():
            pltpu.async_copy(ego_r.at[idx], rbufs[k], gsem.at[k])

    def gather_wait(k):
        pltpu.make_async_copy(ego_l.at[pk_v.at[0, 0]], rbufs[k],
                              gsem.at[k]).wait()

    def scale(j, k):
        rb = rbufs[k]

        def sc16(eb, _):
            vals16 = plsc.bitcast(pk_v[j, 2, pl.ds(eb * 16, 16)], jnp.float32)
            for ek in range(16):
                vv = vals16.at[jnp.full((16,), ek, jnp.int32)].get(
                    mode="promise_in_bounds")
                e = eb * 16 + ek
                rb[e, pl.ds(0, 16)] = rb[e, pl.ds(0, 16)] * vv
                rb[e, pl.ds(16, 16)] = rb[e, pl.ds(16, 16)] * vv
            return 0
        lax.fori_loop(0, CH // 16, sc16, 0)

    if True:
        def scatter_start(j, k):
            pltpu.async_copy(rbufs[k], acc.at[pk_v.at[j, 1]], ssem.at[k],
                             add=True)

        def scatter_wait(k):
            pltpu.make_async_copy(rbufs[k], acc.at[pk_v.at[0, 1]],
                                  ssem.at[k]).wait()

        pltpu.sync_copy(zpage, acc.at[pl.ds(s * SLAB, SLAB)])
        plsc.subcore_barrier()

        def supergroup(g, _):
            base = s * CPT + g * CPS
            pltpu.sync_copy(packed.at[pl.ds(base, CPS)], pk_v)
            for k in range(3):
                gather_start(k, k)

            def group5(jj, _):
                for k in range(NRING):
                    j = jj * NRING + k
                    gather_wait(k)
                    k3 = (k + 3) % NRING
                    if k < 2:
                        @pl.when(jj >= 1)
                        def _():
                            scatter_wait(k3)
                        gather_start(j + 3, k3)
                    else:
                        scatter_wait(k3)

                        @pl.when(jj < CPS // NRING - 1)
                        def _():
                            gather_start(j + 3, k3)
                    scale(j, k)
                    scatter_start(j, k)
                return 0
            lax.fori_loop(0, CPS // NRING, group5, 0)
            scatter_wait(3)
            scatter_wait(4)
            return 0
        lax.fori_loop(0, NSG, supergroup, 0)
        plsc.subcore_barrier()

        @pl.when(c == 0)
        def _():
            pltpu.sync_copy(acc.at[pl.ds(s * SLAB, SLAB)],
                            nb_l.at[pl.ds(s * SLAB, SLAB)])

        @pl.when(c == 1)
        def _():
            pltpu.sync_copy(acc.at[pl.ds(s * SLAB, SLAB)],
                            nb_r.at[pl.ds(s * SLAB, SLAB)])


_BK = 6400


def _dense_body(nbl_ref, nbr_ref, egol_ref, egor_ref,
                wgc_ref, bgc_ref, wbi_ref, bbi_ref,
                outl_ref, outr_ref):
    nb = jnp.concatenate([nbl_ref[...], nbr_ref[...]], axis=1)
    ego = jnp.concatenate([egol_ref[...], egor_ref[...]], axis=1)
    x = (nb @ wgc_ref[...] + bgc_ref[...]
         + (ego * nb) @ wbi_ref[...] + bbi_ref[...])
    h = jnp.where(x >= 0, x, 0.2 * x)
    norm = jnp.maximum(jnp.sqrt(jnp.sum(h * h, axis=1, keepdims=True)), 1e-12)
    o = h / norm
    outl_ref[...] = o[:, :H]
    outr_ref[...] = o[:, H:]


_dense = pl.pallas_call(
    _dense_body,
    grid=(NP // _BK,),
    in_specs=[
        pl.BlockSpec((_BK, H), lambda i: (i, 0)),
        pl.BlockSpec((_BK, H), lambda i: (i, 0)),
        pl.BlockSpec((_BK, H), lambda i: (i, 0)),
        pl.BlockSpec((_BK, H), lambda i: (i, 0)),
        pl.BlockSpec((D, D), lambda i: (0, 0)),
        pl.BlockSpec((1, D), lambda i: (0, 0)),
        pl.BlockSpec((D, D), lambda i: (0, 0)),
        pl.BlockSpec((1, D), lambda i: (0, 0)),
    ],
    out_specs=[
        pl.BlockSpec((_BK, H), lambda i: (i, 0)),
        pl.BlockSpec((_BK, H), lambda i: (i, 0)),
    ],
    out_shape=[
        jax.ShapeDtypeStruct((NP, H), jnp.float32),
        jax.ShapeDtypeStruct((NP, H), jnp.float32),
    ],
)

_GPT = B // (2 * N_SUB)


@functools.partial(
    pl.kernel,
    out_type=(jax.ShapeDtypeStruct((4, 2, B, H), jnp.float32),
              jax.ShapeDtypeStruct((4, 2, B, H), jnp.float32),
              jax.ShapeDtypeStruct((4, 2, B, H), jnp.float32)),
    mesh=_mesh,
    scratch_types=[
        pltpu.VMEM((_GPT,), jnp.int32),
        pltpu.VMEM((_GPT, H), jnp.float32),
    ],
    compiler_params=pltpu.CompilerParams(needs_layout_passes=False,
                                         use_tc_tiling_on_sc=False),
)
def _fgather(l0, r0, l1, r1, l2, r2, l3, r3, iu, ip, ing, ou, op, og,
             idxv, buf):
    c = lax.axis_index("c")
    s = lax.axis_index("s")
    base = (s * 2 + c) * _GPT
    tabs = ((l0, r0), (l1, r1), (l2, r2), (l3, r3))
    for idx_hbm, out in ((iu, ou), (ip, op), (ing, og)):
        pltpu.sync_copy(idx_hbm.at[pl.ds(base, _GPT)], idxv)
        for k in range(4):
            for hh in range(2):
                pltpu.sync_copy(tabs[k][hh].at[idxv], buf)
                pltpu.sync_copy(buf, out.at[k, hh, pl.ds(base, _GPT)])


def kernel(adj_indices, adj_values, users, pos_items, neg_items,
           user_emb, item_emb,
           W_gc_0, b_gc_0, W_bi_0, b_bi_0,
           W_gc_1, b_gc_1, W_bi_1, b_bi_1,
           W_gc_2, b_gc_2, W_bi_2, b_bi_2):
    W_gc = (W_gc_0, W_gc_1, W_gc_2)
    b_gc = (b_gc_0, b_gc_1, b_gc_2)
    W_bi = (W_bi_0, W_bi_1, W_bi_2)
    b_bi = (b_bi_0, b_bi_1, b_bi_2)

    row = adj_indices[0].astype(jnp.int32)
    col = adj_indices[1].astype(jnp.int32)
    rowp = row + jnp.where(row >= N_USERS, P - N_USERS, 0).astype(jnp.int32)
    colp = col + jnp.where(col >= N_USERS, P - N_USERS, 0).astype(jnp.int32)
    vals = adj_values.astype(jnp.float32)
    pad = NNZP - NNZ
    colc = jnp.pad(colp, (0, pad)).reshape(-1, CH)
    rowc = jnp.pad(rowp, (0, pad)).reshape(-1, CH)
    valc = jax.lax.bitcast_convert_type(jnp.pad(vals, (0, pad)),
                                        jnp.int32).reshape(-1, CH)
    packed = jnp.stack([colc, rowc, valc], axis=1)
    zpage = jnp.zeros((SLAB, H), jnp.float32)

    ego_l = jnp.zeros((NP, H), jnp.float32)
    ego_l = ego_l.at[:N_USERS].set(user_emb[:, :H])
    ego_l = ego_l.at[P:P + N_ITEMS].set(item_emb[:, :H])
    ego_r = jnp.zeros((NP, H), jnp.float32)
    ego_r = ego_r.at[:N_USERS].set(user_emb[:, H:])
    ego_r = ego_r.at[P:P + N_ITEMS].set(item_emb[:, H:])

    tabs = [(ego_l, ego_r)]
    for k in range(3):
        nb_l, nb_r = _spmm(ego_l, ego_r, packed, zpage)
        ego_l, ego_r = _dense(nb_l, nb_r, ego_l, ego_r,
                              W_gc[k], b_gc[k], W_bi[k], b_bi[k])
        tabs.append((ego_l, ego_r))

    iu = users.astype(jnp.int32)
    ip = pos_items.astype(jnp.int32) + P
    ig = neg_items.astype(jnp.int32) + P
    ou, opos, oneg = _fgather(tabs[0][0], tabs[0][1], tabs[1][0], tabs[1][1],
                              tabs[2][0], tabs[2][1], tabs[3][0], tabs[3][1],
                              iu, ip, ig)
    u_g = ou.transpose(2, 0, 1, 3).reshape(B, 4 * D)
    pos_g = opos.transpose(2, 0, 1, 3).reshape(B, 4 * D)
    neg_g = oneg.transpose(2, 0, 1, 3).reshape(B, 4 * D)
    return (u_g, pos_g, neg_g)
